# baseline (speedup 1.0000x reference)
import functools

import jax
import jax.numpy as jnp
from jax import lax
from jax.experimental import pallas as pl
from jax.experimental.pallas import tpu as pltpu
from jax.experimental.pallas import tpu_sc as plsc

_B = 4096
_NEG = 20
_D = 64
_NCOL = _NEG + 1

_NC = 2
_NS = 16
_NW = _NC * _NS
_PER_W = _B // _NW
_CHUNK_I = 16
_NCHUNK = _PER_W // _CHUNK_I
_ROWS = _CHUNK_I * _NCOL
_PD = 2 * _D


def _sc_body(tp_hbm, th_hbm, cnp_hbm, cnh_hbm, ttab_hbm, ctab_hbm, dots_hbm,
             idx_tp, th64, idx_cnp, cnh64, t_rows, cn_a, cn_b, out_v,
             sem_t, sem_a, sem_b):
    wid = lax.axis_index("s") * _NC + lax.axis_index("c")
    base = wid * _PER_W

    pltpu.sync_copy(tp_hbm.at[pl.ds(base, _PER_W)], idx_tp)
    pltpu.sync_copy(th_hbm.at[pl.ds(base, _PER_W)], th64)
    pltpu.sync_copy(cnp_hbm.at[pl.ds(base * _NCOL, _PER_W * _NCOL)], idx_cnp)
    pltpu.sync_copy(cnh_hbm.at[pl.ds(base * _NCOL, _PER_W * _NCOL)], cnh64)

    t_cp = pltpu.async_copy(ttab_hbm.at[idx_tp], t_rows, sem_t)

    bufs = (cn_a, cn_b)
    sems = (sem_a, sem_b)

    def gather_chunk(c):
        return pltpu.async_copy(
            ctab_hbm.at[idx_cnp.at[pl.ds(c * _ROWS, _ROWS)]],
            bufs[c % 2], sems[c % 2])

    cur_cp = gather_chunk(0)
    t_cp.wait()

    lanes = lax.iota(jnp.int32, 16)

    for c in range(_NCHUNK):
        nxt_cp = gather_chunk(c + 1) if c + 1 < _NCHUNK else None
        cur_cp.wait()
        buf = bufs[c % 2]
        g = c
        t_row = lanes + g * 16
        t_h = th64[pl.ds(g * 16, 16)]
        cn_row = lanes * _NCOL
        cn_hs = tuple(
            plsc.load_gather(cnh64, [(lanes + g * 16) * _NCOL + jj])
            for jj in range(_NCOL))

        def dbody(d, accs, t_row=t_row, t_h=t_h, cn_row=cn_row,
                  cn_hs=cn_hs, buf=buf):
            t_col = plsc.load_gather(t_rows, [t_row, t_h + d])
            return tuple(
                accs[jj] + t_col * plsc.load_gather(
                    buf, [cn_row + jj, cn_hs[jj] + d])
                for jj in range(_NCOL))

        accs = lax.fori_loop(
            0, _D, dbody,
            tuple(jnp.zeros((16,), jnp.float32) for _ in range(_NCOL)))
        for jj in range(_NCOL):
            out_v[jj, pl.ds(g * 16, 16)] = accs[jj]
        cur_cp = nxt_cp

    pltpu.sync_copy(out_v, dots_hbm.at[wid])


@functools.partial(jax.jit, static_argnames=())
def _sc_dots(t_p, t_h64, cn_p, cn_h64, ttab2, ctab2):
    mesh = plsc.VectorSubcoreMesh(core_axis_name="c", subcore_axis_name="s",
                                  num_cores=_NC, num_subcores=_NS)
    f = pl.kernel(
        _sc_body,
        out_type=jax.ShapeDtypeStruct((_NW, _NCOL, _PER_W), jnp.float32),
        mesh=mesh,
        scratch_types=[
            pltpu.VMEM((_PER_W,), jnp.int32),
            pltpu.VMEM((_PER_W,), jnp.int32),
            pltpu.VMEM((_PER_W * _NCOL,), jnp.int32),
            pltpu.VMEM((_PER_W * _NCOL,), jnp.int32),
            pltpu.VMEM((_PER_W, _PD), jnp.float32),
            pltpu.VMEM((_ROWS, _PD), jnp.float32),
            pltpu.VMEM((_ROWS, _PD), jnp.float32),
            pltpu.VMEM((_NCOL, _PER_W), jnp.float32),
            pltpu.SemaphoreType.DMA,
            pltpu.SemaphoreType.DMA,
            pltpu.SemaphoreType.DMA,
        ],
        compiler_params=pltpu.CompilerParams(needs_layout_passes=False,
                                             use_tc_tiling_on_sc=False),
        name="w2v_ns_dots_sc",
    )
    return f(t_p, t_h64, cn_p, cn_h64, ttab2, ctab2)


def _tc_body(dots_ref, out_ref):
    x = dots_ref[...]
    rows = lax.broadcasted_iota(jnp.int32, x.shape, 0)
    z = jnp.where(rows % _NCOL == 0, -x, x)
    out_ref[0, 0] = jnp.sum(jnp.logaddexp(0.0, z)) * (1.0 / _B)


def _tc_loss(dots):
    return pl.pallas_call(
        _tc_body,
        out_shape=jax.ShapeDtypeStruct((1, 1), jnp.float32),
        out_specs=pl.BlockSpec(memory_space=pltpu.SMEM),
        name="w2v_ns_loss_tc",
    )(dots)[0, 0]


def kernel(target, context, negatives, target_table, context_table):
    target = target.astype(jnp.int32)
    cn_idx = jnp.concatenate(
        [context.reshape(_B, 1).astype(jnp.int32),
         negatives.astype(jnp.int32)], axis=1).reshape(-1)
    t_p = target >> 1
    t_h64 = (target & 1) * _D
    cn_p = cn_idx >> 1
    cn_h64 = (cn_idx & 1) * _D
    ttab2 = target_table.reshape(500000, _PD)
    ctab2 = context_table.reshape(500000, _PD)
    dots = _sc_dots(t_p, t_h64, cn_p, cn_h64, ttab2, ctab2)
    return _tc_loss(dots.reshape(_NW * _NCOL, _PER_W))


# speedup vs baseline: 1.6500x; 1.6500x over previous
import functools

import jax
import jax.numpy as jnp
from jax import lax
from jax.experimental import pallas as pl
from jax.experimental.pallas import tpu as pltpu
from jax.experimental.pallas import tpu_sc as plsc

_B = 4096
_NEG = 20
_D = 64
_NCOL = _NEG + 1

_NC = 2
_NS = 16
_NW = _NC * _NS
_PER_W = _B // _NW
_CHUNK_I = 16
_NCHUNK = _PER_W // _CHUNK_I
_ROWS = _CHUNK_I * _NCOL
_TP = 128


def _sc_t_body(tidx_hbm, ttabT_hbm, tout_hbm,
               slab_a, slab_b, trow_v, idx_v, sem_a, sem_b):
    wid = lax.axis_index("s") * _NC + lax.axis_index("c")
    base = wid * _PER_W

    pltpu.sync_copy(tidx_hbm.at[pl.ds(base, _PER_W)], idx_v)

    slabs = (slab_a, slab_b)
    sems = (sem_a, sem_b)
    lanes = lax.iota(jnp.int32, 16)

    idx_vecs = [idx_v[pl.ds(g * 16, 16)] for g in range(_PER_W // 16)]

    def get_idx(i):
        return idx_vecs[i // 16][i % 16]

    def fetch(i):
        r = get_idx(i)
        off = pl.multiple_of((r >> 7) << 7, 128)
        pltpu.async_copy(ttabT_hbm.at[:, pl.ds(off, 128)],
                         slabs[i % 2], sems[i % 2])

    def wait_slab(i):
        pltpu.make_async_copy(ttabT_hbm.at[:, pl.ds(0, 128)],
                              slabs[i % 2], sems[i % 2]).wait()

    fetch(0)
    for i in range(_PER_W):
        if i + 1 < _PER_W:
            fetch(i + 1)
        wait_slab(i)
        col = jnp.full((16,), get_idx(i) & 127, dtype=jnp.int32)
        for k in range(_D // 16):
            vals = plsc.load_gather(slabs[i % 2], [lanes + k * 16, col])
            trow_v[i, pl.ds(k * 16, 16)] = vals

    pltpu.sync_copy(trow_v,
                    tout_hbm.at[pl.ds(pl.multiple_of(base, 128), _PER_W)])


@functools.partial(jax.jit, static_argnames=())
def _sc_trows(target, ttab_t):
    mesh = plsc.VectorSubcoreMesh(core_axis_name="c", subcore_axis_name="s",
                                  num_cores=_NC, num_subcores=_NS)
    f = pl.kernel(
        _sc_t_body,
        out_type=jax.ShapeDtypeStruct((_B, _TP), jnp.float32),
        mesh=mesh,
        scratch_types=[
            pltpu.VMEM((_D, 128), jnp.float32),
            pltpu.VMEM((_D, 128), jnp.float32),
            pltpu.VMEM((_PER_W, _TP), jnp.float32),
            pltpu.VMEM((_PER_W,), jnp.int32),
            pltpu.SemaphoreType.DMA,
            pltpu.SemaphoreType.DMA,
        ],
        compiler_params=pltpu.CompilerParams(needs_layout_passes=False,
                                             use_tc_tiling_on_sc=True),
        name="w2v_ns_trows_sc",
    )
    return f(target, ttab_t)


def _sc_dots_body(cnidx_hbm, trows_hbm, ctab_hbm, dots_hbm,
                  t_v, cn_a, cn_b, out_v, idx_cn, sem_t, sem_a, sem_b):
    wid = lax.axis_index("s") * _NC + lax.axis_index("c")
    base = wid * _PER_W

    pltpu.sync_copy(cnidx_hbm.at[pl.ds(base * _NCOL, _PER_W * _NCOL)], idx_cn)
    t_cp = pltpu.async_copy(
        trows_hbm.at[pl.ds(pl.multiple_of(base, 128), _PER_W)], t_v, sem_t)

    bufs = (cn_a, cn_b)
    sems = (sem_a, sem_b)

    def gather_chunk(c):
        return pltpu.async_copy(
            ctab_hbm.at[idx_cn.at[pl.ds(c * _ROWS, _ROWS)]],
            bufs[c % 2], sems[c % 2])

    cur_cp = gather_chunk(0)
    t_cp.wait()

    lanes = lax.iota(jnp.int32, 16)

    for c in range(_NCHUNK):
        nxt_cp = gather_chunk(c + 1) if c + 1 < _NCHUNK else None
        cur_cp.wait()
        buf = bufs[c % 2]
        g = c
        t_row = lanes + g * 16
        cn_row = lanes * _NCOL

        def dbody(d, accs, t_row=t_row, cn_row=cn_row, buf=buf):
            dcol = jnp.full((16,), d, dtype=jnp.int32)
            t_col = plsc.load_gather(t_v, [t_row, dcol])
            return tuple(
                accs[jj] + t_col * plsc.load_gather(
                    buf, [cn_row + jj, dcol])
                for jj in range(_NCOL))

        accs = lax.fori_loop(
            0, _D, dbody,
            tuple(jnp.zeros((16,), jnp.float32) for _ in range(_NCOL)))
        for jj in range(_NCOL):
            out_v[jj, pl.ds(g * 16, 16)] = accs[jj]
        cur_cp = nxt_cp

    pltpu.sync_copy(out_v, dots_hbm.at[wid])


@functools.partial(jax.jit, static_argnames=())
def _sc_dots(cn_idx, t_rows, context_table):
    mesh = plsc.VectorSubcoreMesh(core_axis_name="c", subcore_axis_name="s",
                                  num_cores=_NC, num_subcores=_NS)
    f = pl.kernel(
        _sc_dots_body,
        out_type=jax.ShapeDtypeStruct((_NW, _NCOL, _PER_W), jnp.float32),
        mesh=mesh,
        scratch_types=[
            pltpu.VMEM((_PER_W, _TP), jnp.float32),
            pltpu.VMEM((_ROWS, _D), jnp.float32),
            pltpu.VMEM((_ROWS, _D), jnp.float32),
            pltpu.VMEM((_NCOL, _PER_W), jnp.float32),
            pltpu.VMEM((_PER_W * _NCOL,), jnp.int32),
            pltpu.SemaphoreType.DMA,
            pltpu.SemaphoreType.DMA,
            pltpu.SemaphoreType.DMA,
        ],
        compiler_params=pltpu.CompilerParams(needs_layout_passes=False,
                                             use_tc_tiling_on_sc=False),
        name="w2v_ns_dots_sc",
    )
    return f(cn_idx, t_rows, context_table)


def _tc_body(dots_ref, out_ref):
    x = dots_ref[...]
    cols = lax.broadcasted_iota(jnp.int32, x.shape, 1)
    z = jnp.where(cols == 0, -x, x)
    out_ref[0, 0] = jnp.sum(jnp.logaddexp(0.0, z)) * (1.0 / _B)


def _tc_loss(dots):
    return pl.pallas_call(
        _tc_body,
        out_shape=jax.ShapeDtypeStruct((1, 1), jnp.float32),
        out_specs=pl.BlockSpec(memory_space=pltpu.SMEM),
        name="w2v_ns_loss_tc",
    )(dots)[0, 0]


def kernel(target, context, negatives, target_table, context_table):
    cn_idx = jnp.concatenate(
        [context.reshape(_B, 1).astype(jnp.int32),
         negatives.astype(jnp.int32)], axis=1).reshape(-1)
    t_rows = _sc_trows(target.astype(jnp.int32), target_table.T)
    dots = _sc_dots(cn_idx, t_rows, context_table)
    return _tc_loss(dots)
